# Initial kernel scaffold; baseline (speedup 1.0000x reference)
import jax
import jax.numpy as jnp
from jax.experimental import pallas as pl


def kernel(x, edge_index, W, b):
    raise NotImplementedError("write your pallas kernel here")


# speedup vs baseline: 21.2907x; 21.2907x over previous
import functools

import jax
import jax.numpy as jnp
from jax import lax
from jax.experimental import pallas as pl
from jax.experimental.pallas import tpu as pltpu
from jax.experimental.pallas import tpu_sc as plsc

N = 10000
NPAD = 10240
E = 320000
D = 128

NC = 2
NS = 16
NW = NC * NS

C = 80
EPW = E // NW
NCH = EPW // C
EPT = E // NS
NCHD = EPT // C
NPW = NPAD // NW

_mesh = plsc.VectorSubcoreMesh(core_axis_name="c", subcore_axis_name="s")
_sc_params = pltpu.CompilerParams(needs_layout_passes=False)


def _prep_body(rowp_hbm, x_hbm, dis_hbm, y_hbm,
               deg_sh, rowv, onesv, zdeg, dgv, disv, xv):
    c = lax.axis_index("c")
    s = lax.axis_index("s")
    for j in range(C // 16):
        onesv[pl.ds(16 * j, 16)] = jnp.ones((16,), jnp.float32)
    for j in range(40):
        zdeg[pl.ds(16 * j, 16)] = jnp.zeros((16,), jnp.float32)
    pltpu.sync_copy(zdeg, deg_sh.at[pl.ds(640 * s, 640)])
    plsc.subcore_barrier()
    pltpu.sync_copy(rowp_hbm.at[s], rowv)
    for i in range(NCHD):
        pltpu.sync_copy(onesv, deg_sh.at[rowv.at[i]], add=True)
    plsc.subcore_barrier()
    nb = 5120 * c + 320 * s
    pltpu.sync_copy(deg_sh.at[pl.ds(nb, NPW)], dgv)
    for j in range(NPW // 16):
        dg = dgv[pl.ds(16 * j, 16)]
        dgc = jnp.maximum(dg, 1.0)
        u = lax.bitcast_convert_type(dgc, jnp.int32)
        u = jnp.int32(0x5F3759DF) - (u >> 1)
        h = lax.bitcast_convert_type(u, jnp.float32)
        for _ in range(3):
            h = h * (1.5 - 0.5 * dgc * h * h)
        disv[pl.ds(16 * j, 16)] = jnp.where(dg > 0.0, h, 0.0)
    pltpu.sync_copy(disv, dis_hbm.at[pl.ds(nb, NPW)])
    pltpu.sync_copy(x_hbm.at[pl.ds(nb, NPW)], xv)

    def nbody(n, carry):
        idxn = jnp.zeros((16,), jnp.int32) + n
        s16 = plsc.load_gather(disv, [idxn])
        for j in range(D // 16):
            xv[n, pl.ds(16 * j, 16)] = xv[n, pl.ds(16 * j, 16)] * s16
        return carry

    lax.fori_loop(0, NPW, nbody, 0)
    pltpu.sync_copy(xv, y_hbm.at[pl.ds(nb, NPW)])


@functools.partial(
    pl.kernel,
    out_type=(
        jax.ShapeDtypeStruct((NPAD,), jnp.float32),
        jax.ShapeDtypeStruct((NPAD, D), jnp.float32),
    ),
    mesh=_mesh,
    scratch_types=[
        pltpu.VMEM_SHARED((NPAD,), jnp.float32),
        pltpu.VMEM((NCHD, C), jnp.int32),
        pltpu.VMEM((C,), jnp.float32),
        pltpu.VMEM((640,), jnp.float32),
        pltpu.VMEM((NPW,), jnp.float32),
        pltpu.VMEM((NPW,), jnp.float32),
        pltpu.VMEM((NPW, D), jnp.float32),
    ],
    compiler_params=_sc_params,
)
def _prep(*args):
    _prep_body(*args)


def _agg_body(row3_hbm, col3_hbm, y_hbm, z2_hbm,
              acc_sh, rowv, colv, ybuf, tmp):
    c = lax.axis_index("c")
    s = lax.axis_index("s")
    w = s * NC + c

    def zbody(k, carry):
        for j in range(D // 16):
            tmp[k, pl.ds(16 * j, 16)] = jnp.zeros((16,), jnp.float32)
        return carry

    lax.fori_loop(0, 16, zbody, 0)
    for k in range(40):
        pltpu.sync_copy(tmp, acc_sh.at[pl.ds(640 * s + 16 * k, 16)])
    plsc.subcore_barrier()
    pltpu.sync_copy(row3_hbm.at[w], rowv)
    pltpu.sync_copy(col3_hbm.at[w], colv)
    for i in range(NCH):
        pltpu.sync_copy(y_hbm.at[colv.at[i]], ybuf)
        pltpu.sync_copy(ybuf, acc_sh.at[rowv.at[i]], add=True)
    plsc.subcore_barrier()
    for k in range(40):
        pltpu.sync_copy(acc_sh.at[pl.ds(640 * s + 16 * k, 16)], tmp)
        pltpu.sync_copy(tmp, z2_hbm.at[c, pl.ds(640 * s + 16 * k, 16)])


@functools.partial(
    pl.kernel,
    out_type=jax.ShapeDtypeStruct((NC, NPAD, D), jnp.float32),
    mesh=_mesh,
    scratch_types=[
        pltpu.VMEM_SHARED((NPAD, D), jnp.float32),
        pltpu.VMEM((NCH, C), jnp.int32),
        pltpu.VMEM((NCH, C), jnp.int32),
        pltpu.VMEM((C, D), jnp.float32),
        pltpu.VMEM((16, D), jnp.float32),
    ],
    compiler_params=_sc_params,
)
def _agg(*args):
    _agg_body(*args)


def _mm_body(z2_ref, dis_ref, w_ref, b_ref, o_ref):
    z = z2_ref[0] + z2_ref[1]
    zd = z * dis_ref[...]
    r = lax.dot_general(
        zd, w_ref[...], (((1,), (1,)), ((), ())),
        preferred_element_type=jnp.float32,
        precision=lax.Precision.HIGHEST,
    )
    o_ref[...] = r + b_ref[...]


def _mm(z2, dis_col, W, b2):
    blk = 256
    grid = NPAD // blk
    return pl.pallas_call(
        _mm_body,
        grid=(grid,),
        in_specs=[
            pl.BlockSpec((NC, blk, D), lambda i: (0, i, 0)),
            pl.BlockSpec((blk, 1), lambda i: (i, 0)),
            pl.BlockSpec((D, D), lambda i: (0, 0)),
            pl.BlockSpec((1, D), lambda i: (0, 0)),
        ],
        out_specs=pl.BlockSpec((blk, D), lambda i: (i, 0)),
        out_shape=jax.ShapeDtypeStruct((NPAD, D), jnp.float32),
    )(z2, dis_col, W, b2)


@jax.jit
def kernel(x, edge_index, W, b):
    row = edge_index[0].astype(jnp.int32)
    col = edge_index[1].astype(jnp.int32)
    rowp = row.reshape(NS, NCHD, C)
    row3 = row.reshape(NW, NCH, C)
    col3 = col.reshape(NW, NCH, C)
    x_pad = jnp.pad(x, ((0, NPAD - N), (0, 0)))
    dis, y = _prep(rowp, x_pad)
    z2 = _agg(row3, col3, y)
    out = _mm(z2, dis.reshape(NPAD, 1), W, b.reshape(1, D))
    return out[:N]


# speedup vs baseline: 30.0998x; 1.4138x over previous
import functools

import jax
import jax.numpy as jnp
from jax import lax
from jax.experimental import pallas as pl
from jax.experimental.pallas import tpu as pltpu
from jax.experimental.pallas import tpu_sc as plsc

N = 10000
NPAD = 10240
E = 320000
D = 128

NC = 2
NS = 16
NW = NC * NS

C = 80
EPW = E // NW
NCH = EPW // C
EPT = E // NS
NCHD = EPT // C
NPW = NPAD // NW

_mesh = plsc.VectorSubcoreMesh(core_axis_name="c", subcore_axis_name="s")
_sc_params = pltpu.CompilerParams(needs_layout_passes=False)


def _prep_body(rowp_hbm, x_hbm, dis_hbm, y_hbm,
               deg_sh, rowv, onesv, zdeg, dgv, disv, xv, dsem):
    c = lax.axis_index("c")
    s = lax.axis_index("s")
    for j in range(C // 16):
        onesv[pl.ds(16 * j, 16)] = jnp.ones((16,), jnp.float32)
    for j in range(40):
        zdeg[pl.ds(16 * j, 16)] = jnp.zeros((16,), jnp.float32)
    pltpu.sync_copy(zdeg, deg_sh.at[pl.ds(640 * s, 640)])
    plsc.subcore_barrier()
    pltpu.sync_copy(rowp_hbm.at[s], rowv)
    for grp in range(NCHD // 25):
        descs = [
            pltpu.async_copy(onesv, deg_sh.at[rowv.at[25 * grp + k]],
                             dsem, add=True)
            for k in range(25)
        ]
        for d in descs:
            d.wait()
    plsc.subcore_barrier()
    nb = 5120 * c + 320 * s
    pltpu.sync_copy(deg_sh.at[pl.ds(nb, NPW)], dgv)
    for j in range(NPW // 16):
        dg = dgv[pl.ds(16 * j, 16)]
        dgc = jnp.maximum(dg, 1.0)
        u = lax.bitcast_convert_type(dgc, jnp.int32)
        u = jnp.int32(0x5F3759DF) - (u >> 1)
        h = lax.bitcast_convert_type(u, jnp.float32)
        for _ in range(3):
            h = h * (1.5 - 0.5 * dgc * h * h)
        disv[pl.ds(16 * j, 16)] = jnp.where(dg > 0.0, h, 0.0)
    pltpu.sync_copy(disv, dis_hbm.at[pl.ds(nb, NPW)])
    pltpu.sync_copy(x_hbm.at[pl.ds(nb, NPW)], xv)

    def nbody(n, carry):
        idxn = jnp.zeros((16,), jnp.int32) + n
        s16 = plsc.load_gather(disv, [idxn])
        for j in range(D // 16):
            xv[n, pl.ds(16 * j, 16)] = xv[n, pl.ds(16 * j, 16)] * s16
        return carry

    lax.fori_loop(0, NPW, nbody, 0)
    pltpu.sync_copy(xv, y_hbm.at[pl.ds(nb, NPW)])


@functools.partial(
    pl.kernel,
    out_type=(
        jax.ShapeDtypeStruct((NPAD,), jnp.float32),
        jax.ShapeDtypeStruct((NPAD, D), jnp.float32),
    ),
    mesh=_mesh,
    scratch_types=[
        pltpu.VMEM_SHARED((NPAD,), jnp.float32),
        pltpu.VMEM((NCHD, C), jnp.int32),
        pltpu.VMEM((C,), jnp.float32),
        pltpu.VMEM((640,), jnp.float32),
        pltpu.VMEM((NPW,), jnp.float32),
        pltpu.VMEM((NPW,), jnp.float32),
        pltpu.VMEM((NPW, D), jnp.float32),
        pltpu.SemaphoreType.DMA,
    ],
    compiler_params=_sc_params,
)
def _prep(*args):
    _prep_body(*args)


BLK = 25
NBLK = NCH // BLK


def _agg_body(row3_hbm, col3_hbm, y_hbm, z2_hbm,
              acc_sh, rowv, colv, gbuf, tmp, gsem):
    c = lax.axis_index("c")
    s = lax.axis_index("s")
    w = s * NC + c

    def zbody(k, carry):
        for j in range(D // 16):
            tmp[k, pl.ds(16 * j, 16)] = jnp.zeros((16,), jnp.float32)
        return carry

    lax.fori_loop(0, 16, zbody, 0)
    for k in range(40):
        pltpu.sync_copy(tmp, acc_sh.at[pl.ds(640 * s + 16 * k, 16)])
    plsc.subcore_barrier()
    for blk in range(NBLK):
        pltpu.sync_copy(row3_hbm.at[w, blk], rowv)
        pltpu.sync_copy(col3_hbm.at[w, blk], colv)
        g = pltpu.async_copy(y_hbm.at[colv.at[0]], gbuf.at[0], gsem)
        for t in range(BLK):
            gn = None
            if t + 1 < BLK:
                gn = pltpu.async_copy(
                    y_hbm.at[colv.at[t + 1]], gbuf.at[(t + 1) % 2], gsem)
            g.wait()
            pltpu.sync_copy(gbuf.at[t % 2], acc_sh.at[rowv.at[t]], add=True)
            g = gn
    plsc.subcore_barrier()
    for k in range(40):
        pltpu.sync_copy(acc_sh.at[pl.ds(640 * s + 16 * k, 16)], tmp)
        pltpu.sync_copy(tmp, z2_hbm.at[c, pl.ds(640 * s + 16 * k, 16)])


@functools.partial(
    pl.kernel,
    out_type=jax.ShapeDtypeStruct((NC, NPAD, D), jnp.float32),
    mesh=_mesh,
    scratch_types=[
        pltpu.VMEM_SHARED((NPAD, D), jnp.float32),
        pltpu.VMEM((BLK, C), jnp.int32),
        pltpu.VMEM((BLK, C), jnp.int32),
        pltpu.VMEM((2, C, D), jnp.float32),
        pltpu.VMEM((16, D), jnp.float32),
        pltpu.SemaphoreType.DMA,
    ],
    compiler_params=_sc_params,
)
def _agg(*args):
    _agg_body(*args)


def _mm_body(z2_ref, dis_ref, w_ref, b_ref, o_ref):
    z = z2_ref[0] + z2_ref[1]
    zd = z * dis_ref[...]
    r = lax.dot_general(
        zd, w_ref[...], (((1,), (1,)), ((), ())),
        preferred_element_type=jnp.float32,
        precision=lax.Precision.HIGHEST,
    )
    o_ref[...] = r + b_ref[...]


def _mm(z2, dis_col, W, b2):
    blk = 256
    grid = NPAD // blk
    return pl.pallas_call(
        _mm_body,
        grid=(grid,),
        in_specs=[
            pl.BlockSpec((NC, blk, D), lambda i: (0, i, 0)),
            pl.BlockSpec((blk, 1), lambda i: (i, 0)),
            pl.BlockSpec((D, D), lambda i: (0, 0)),
            pl.BlockSpec((1, D), lambda i: (0, 0)),
        ],
        out_specs=pl.BlockSpec((blk, D), lambda i: (i, 0)),
        out_shape=jax.ShapeDtypeStruct((NPAD, D), jnp.float32),
    )(z2, dis_col, W, b2)


@jax.jit
def kernel(x, edge_index, W, b):
    row = edge_index[0].astype(jnp.int32)
    col = edge_index[1].astype(jnp.int32)
    rowp = row.reshape(NS, NCHD, C)
    row3 = row.reshape(NW, NBLK, BLK, C)
    col3 = col.reshape(NW, NBLK, BLK, C)
    x_pad = jnp.pad(x, ((0, NPAD - N), (0, 0)))
    dis, y = _prep(rowp, x_pad)
    z2 = _agg(row3, col3, y)
    out = _mm(z2, dis.reshape(NPAD, 1), W, b.reshape(1, D))
    return out[:N]


# speedup vs baseline: 34.9856x; 1.1623x over previous
import functools

import jax
import jax.numpy as jnp
from jax import lax
from jax.experimental import pallas as pl
from jax.experimental.pallas import tpu as pltpu
from jax.experimental.pallas import tpu_sc as plsc

N = 10000
NPAD = 10240
E = 320000
D = 128

NC = 2
NS = 16
NW = NC * NS

C = 80
EPW = E // NW
NCH = EPW // C
EPT = E // NS
NCHD = EPT // C
NPW = NPAD // NW

_mesh = plsc.VectorSubcoreMesh(core_axis_name="c", subcore_axis_name="s")
_sc_params = pltpu.CompilerParams(needs_layout_passes=False)


def _prep_body(rowp_hbm, x_hbm, dis_hbm, y_hbm,
               deg_sh, rowv, onesv, zdeg, dgv, disv, xv, dsem):
    c = lax.axis_index("c")
    s = lax.axis_index("s")
    for j in range(C // 16):
        onesv[pl.ds(16 * j, 16)] = jnp.ones((16,), jnp.float32)
    for j in range(40):
        zdeg[pl.ds(16 * j, 16)] = jnp.zeros((16,), jnp.float32)
    pltpu.sync_copy(zdeg, deg_sh.at[pl.ds(640 * s, 640)])
    plsc.subcore_barrier()
    pltpu.sync_copy(rowp_hbm.at[s], rowv)
    for grp in range(NCHD // 25):
        descs = [
            pltpu.async_copy(onesv, deg_sh.at[rowv.at[25 * grp + k]],
                             dsem, add=True)
            for k in range(25)
        ]
        for d in descs:
            d.wait()
    plsc.subcore_barrier()
    nb = 5120 * c + 320 * s
    pltpu.sync_copy(deg_sh.at[pl.ds(nb, NPW)], dgv)
    for j in range(NPW // 16):
        dg = dgv[pl.ds(16 * j, 16)]
        dgc = jnp.maximum(dg, 1.0)
        u = lax.bitcast_convert_type(dgc, jnp.int32)
        u = jnp.int32(0x5F3759DF) - (u >> 1)
        h = lax.bitcast_convert_type(u, jnp.float32)
        for _ in range(3):
            h = h * (1.5 - 0.5 * dgc * h * h)
        disv[pl.ds(16 * j, 16)] = jnp.where(dg > 0.0, h, 0.0)
    pltpu.sync_copy(disv, dis_hbm.at[pl.ds(nb, NPW)])
    pltpu.sync_copy(x_hbm.at[pl.ds(nb, NPW)], xv)

    def nbody(n, carry):
        idxn = jnp.zeros((16,), jnp.int32) + n
        s16 = plsc.load_gather(disv, [idxn])
        for j in range(D // 16):
            xv[n, pl.ds(16 * j, 16)] = xv[n, pl.ds(16 * j, 16)] * s16
        return carry

    lax.fori_loop(0, NPW, nbody, 0)
    pltpu.sync_copy(xv, y_hbm.at[pl.ds(nb, NPW)])


@functools.partial(
    pl.kernel,
    out_type=(
        jax.ShapeDtypeStruct((NPAD,), jnp.float32),
        jax.ShapeDtypeStruct((NPAD, D), jnp.float32),
    ),
    mesh=_mesh,
    scratch_types=[
        pltpu.VMEM_SHARED((NPAD,), jnp.float32),
        pltpu.VMEM((NCHD, C), jnp.int32),
        pltpu.VMEM((C,), jnp.float32),
        pltpu.VMEM((640,), jnp.float32),
        pltpu.VMEM((NPW,), jnp.float32),
        pltpu.VMEM((NPW,), jnp.float32),
        pltpu.VMEM((NPW, D), jnp.float32),
        pltpu.SemaphoreType.DMA,
    ],
    compiler_params=_sc_params,
)
def _prep(*args):
    _prep_body(*args)


BLK = 25
NBLK = NCH // BLK


def _agg_body(row3_hbm, col3_hbm, y_hbm, z2_hbm,
              acc_sh, rowv, colv, gbuf, tmp, gsem, ssem, zsem):
    c = lax.axis_index("c")
    s = lax.axis_index("s")
    w = s * NC + c

    def zbody(k, carry):
        for j in range(D // 16):
            tmp[k, pl.ds(16 * j, 16)] = jnp.zeros((16,), jnp.float32)
        return carry

    lax.fori_loop(0, 16, zbody, 0)
    zd = [
        pltpu.async_copy(tmp, acc_sh.at[pl.ds(640 * s + 16 * k, 16)], zsem)
        for k in range(40)
    ]
    for d in zd:
        d.wait()
    plsc.subcore_barrier()
    for blk in range(NBLK):
        pltpu.sync_copy(row3_hbm.at[w, blk], rowv)
        pltpu.sync_copy(col3_hbm.at[w, blk], colv)
        g = [None] * BLK
        sc = [None] * BLK
        g[0] = pltpu.async_copy(y_hbm.at[colv.at[0]], gbuf.at[0], gsem)
        for t in range(BLK):
            if t >= 2:
                sc[t - 2].wait()
            if t + 1 < BLK:
                g[t + 1] = pltpu.async_copy(
                    y_hbm.at[colv.at[t + 1]], gbuf.at[(t + 1) % 3], gsem)
            g[t].wait()
            sc[t] = pltpu.async_copy(
                gbuf.at[t % 3], acc_sh.at[rowv.at[t]], ssem, add=True)
        sc[BLK - 2].wait()
        sc[BLK - 1].wait()
    plsc.subcore_barrier()
    pltpu.sync_copy(acc_sh.at[pl.ds(640 * s, 640)],
                    z2_hbm.at[c, pl.ds(640 * s, 640)])


@functools.partial(
    pl.kernel,
    out_type=jax.ShapeDtypeStruct((NC, NPAD, D), jnp.float32),
    mesh=_mesh,
    scratch_types=[
        pltpu.VMEM_SHARED((NPAD, D), jnp.float32),
        pltpu.VMEM((BLK, C), jnp.int32),
        pltpu.VMEM((BLK, C), jnp.int32),
        pltpu.VMEM((3, C, D), jnp.float32),
        pltpu.VMEM((16, D), jnp.float32),
        pltpu.SemaphoreType.DMA,
        pltpu.SemaphoreType.DMA,
        pltpu.SemaphoreType.DMA,
    ],
    compiler_params=_sc_params,
)
def _agg(*args):
    _agg_body(*args)


def _mm_body(z2_ref, dis_ref, w_ref, b_ref, o_ref):
    z = z2_ref[0] + z2_ref[1]
    zd = z * dis_ref[...]
    r = lax.dot_general(
        zd, w_ref[...], (((1,), (1,)), ((), ())),
        preferred_element_type=jnp.float32,
        precision=lax.Precision.HIGHEST,
    )
    o_ref[...] = r + b_ref[...]


def _mm(z2, dis_col, W, b2):
    blk = 256
    grid = NPAD // blk
    return pl.pallas_call(
        _mm_body,
        grid=(grid,),
        in_specs=[
            pl.BlockSpec((NC, blk, D), lambda i: (0, i, 0)),
            pl.BlockSpec((blk, 1), lambda i: (i, 0)),
            pl.BlockSpec((D, D), lambda i: (0, 0)),
            pl.BlockSpec((1, D), lambda i: (0, 0)),
        ],
        out_specs=pl.BlockSpec((blk, D), lambda i: (i, 0)),
        out_shape=jax.ShapeDtypeStruct((N, D), jnp.float32),
    )(z2, dis_col, W, b2)


@jax.jit
def kernel(x, edge_index, W, b):
    row = edge_index[0].astype(jnp.int32)
    col = edge_index[1].astype(jnp.int32)
    rowp = row.reshape(NS, NCHD, C)
    row3 = row.reshape(NW, NBLK, BLK, C)
    col3 = col.reshape(NW, NBLK, BLK, C)
    x_pad = jnp.pad(x, ((0, NPAD - N), (0, 0)))
    dis, y = _prep(rowp, x_pad)
    z2 = _agg(row3, col3, y)
    return _mm(z2, dis.reshape(NPAD, 1), W, b.reshape(1, D))


# speedup vs baseline: 36.0544x; 1.0305x over previous
import functools

import jax
import jax.numpy as jnp
from jax import lax
from jax.experimental import pallas as pl
from jax.experimental.pallas import tpu as pltpu
from jax.experimental.pallas import tpu_sc as plsc

N = 10000
NPAD = 10240
E = 320000
D = 128

NC = 2
NS = 16
NW = NC * NS

C = 80
EPW = E // NW
NCH = EPW // C
EPT = E // NS
NCHD = EPT // C
NPW = NPAD // NW

_mesh = plsc.VectorSubcoreMesh(core_axis_name="c", subcore_axis_name="s")
_sc_params = pltpu.CompilerParams(needs_layout_passes=False)


def _prep_body(rowp_hbm, x_hbm, dis_hbm, y_hbm,
               deg_sh, rowv, onesv, zdeg, dgv, disv, xv, dsem):
    c = lax.axis_index("c")
    s = lax.axis_index("s")
    for j in range(C // 16):
        onesv[pl.ds(16 * j, 16)] = jnp.ones((16,), jnp.float32)
    for j in range(40):
        zdeg[pl.ds(16 * j, 16)] = jnp.zeros((16,), jnp.float32)
    pltpu.sync_copy(zdeg, deg_sh.at[pl.ds(640 * s, 640)])
    plsc.subcore_barrier()
    pltpu.sync_copy(rowp_hbm.at[s], rowv)
    for grp in range(NCHD // 25):
        descs = [
            pltpu.async_copy(onesv, deg_sh.at[rowv.at[25 * grp + k]],
                             dsem, add=True)
            for k in range(25)
        ]
        for d in descs:
            d.wait()
    plsc.subcore_barrier()
    nb = 5120 * c + 320 * s
    pltpu.sync_copy(deg_sh.at[pl.ds(nb, NPW)], dgv)
    for j in range(NPW // 16):
        dg = dgv[pl.ds(16 * j, 16)]
        dgc = jnp.maximum(dg, 1.0)
        u = lax.bitcast_convert_type(dgc, jnp.int32)
        u = jnp.int32(0x5F3759DF) - (u >> 1)
        h = lax.bitcast_convert_type(u, jnp.float32)
        for _ in range(3):
            h = h * (1.5 - 0.5 * dgc * h * h)
        disv[pl.ds(16 * j, 16)] = jnp.where(dg > 0.0, h, 0.0)
    pltpu.sync_copy(disv, dis_hbm.at[pl.ds(nb, NPW)])
    pltpu.sync_copy(x_hbm.at[pl.ds(nb, NPW)], xv)

    def nbody(n, carry):
        idxn = jnp.zeros((16,), jnp.int32) + n
        s16 = plsc.load_gather(disv, [idxn])
        for j in range(D // 16):
            xv[n, pl.ds(16 * j, 16)] = xv[n, pl.ds(16 * j, 16)] * s16
        return carry

    lax.fori_loop(0, NPW, nbody, 0)
    pltpu.sync_copy(xv, y_hbm.at[pl.ds(nb, NPW)])


@functools.partial(
    pl.kernel,
    out_type=(
        jax.ShapeDtypeStruct((NPAD,), jnp.float32),
        jax.ShapeDtypeStruct((NPAD, D), jnp.float32),
    ),
    mesh=_mesh,
    scratch_types=[
        pltpu.VMEM_SHARED((NPAD,), jnp.float32),
        pltpu.VMEM((NCHD, C), jnp.int32),
        pltpu.VMEM((C,), jnp.float32),
        pltpu.VMEM((640,), jnp.float32),
        pltpu.VMEM((NPW,), jnp.float32),
        pltpu.VMEM((NPW,), jnp.float32),
        pltpu.VMEM((NPW, D), jnp.float32),
        pltpu.SemaphoreType.DMA,
    ],
    compiler_params=_sc_params,
)
def _prep(*args):
    _prep_body(*args)


BLK = 25
NBLK = NCH // BLK


def _agg_body(row3_hbm, col3_hbm, y_hbm, z2_hbm,
              acc_sh, rowv, colv, gbuf, gsem, ssem, zsem):
    c = lax.axis_index("c")
    s = lax.axis_index("s")
    w = s * NC + c

    def zbody(k, carry):
        for j in range(D // 16):
            gbuf[0, k, pl.ds(16 * j, 16)] = jnp.zeros((16,), jnp.float32)
        return carry

    lax.fori_loop(0, C, zbody, 0)
    zd = [
        pltpu.async_copy(gbuf.at[0], acc_sh.at[pl.ds(640 * s + C * k, C)], zsem)
        for k in range(640 // C)
    ]
    for d in zd:
        d.wait()
    plsc.subcore_barrier()
    for blk in range(NBLK):
        pltpu.sync_copy(row3_hbm.at[w, blk], rowv)
        pltpu.sync_copy(col3_hbm.at[w, blk], colv)
        g = [None] * BLK
        sc = [None] * BLK
        g[0] = pltpu.async_copy(y_hbm.at[colv.at[0]], gbuf.at[0], gsem)
        g[1] = pltpu.async_copy(y_hbm.at[colv.at[1]], gbuf.at[1], gsem)
        for t in range(BLK):
            if t >= 2:
                sc[t - 2].wait()
            if t + 2 < BLK:
                g[t + 2] = pltpu.async_copy(
                    y_hbm.at[colv.at[t + 2]], gbuf.at[(t + 2) % 4], gsem)
            g[t].wait()
            sc[t] = pltpu.async_copy(
                gbuf.at[t % 4], acc_sh.at[rowv.at[t]], ssem, add=True)
        sc[BLK - 2].wait()
        sc[BLK - 1].wait()
    plsc.subcore_barrier()
    pltpu.sync_copy(acc_sh.at[pl.ds(640 * s, 640)],
                    z2_hbm.at[c, pl.ds(640 * s, 640)])


@functools.partial(
    pl.kernel,
    out_type=jax.ShapeDtypeStruct((NC, NPAD, D), jnp.float32),
    mesh=_mesh,
    scratch_types=[
        pltpu.VMEM_SHARED((NPAD, D), jnp.float32),
        pltpu.VMEM((BLK, C), jnp.int32),
        pltpu.VMEM((BLK, C), jnp.int32),
        pltpu.VMEM((4, C, D), jnp.float32),
        pltpu.SemaphoreType.DMA,
        pltpu.SemaphoreType.DMA,
        pltpu.SemaphoreType.DMA,
    ],
    compiler_params=_sc_params,
)
def _agg(*args):
    _agg_body(*args)


def _mm_body(z2_ref, dis_ref, w_ref, b_ref, o_ref):
    z = z2_ref[0] + z2_ref[1]
    zd = z * dis_ref[...]
    r = lax.dot_general(
        zd, w_ref[...], (((1,), (1,)), ((), ())),
        preferred_element_type=jnp.float32,
        precision=lax.Precision.HIGHEST,
    )
    o_ref[...] = r + b_ref[...]


def _mm(z2, dis_col, W, b2):
    blk = 256
    grid = NPAD // blk
    return pl.pallas_call(
        _mm_body,
        grid=(grid,),
        in_specs=[
            pl.BlockSpec((NC, blk, D), lambda i: (0, i, 0)),
            pl.BlockSpec((blk, 1), lambda i: (i, 0)),
            pl.BlockSpec((D, D), lambda i: (0, 0)),
            pl.BlockSpec((1, D), lambda i: (0, 0)),
        ],
        out_specs=pl.BlockSpec((blk, D), lambda i: (i, 0)),
        out_shape=jax.ShapeDtypeStruct((N, D), jnp.float32),
    )(z2, dis_col, W, b2)


@jax.jit
def kernel(x, edge_index, W, b):
    row = edge_index[0].astype(jnp.int32)
    col = edge_index[1].astype(jnp.int32)
    rowp = row.reshape(NS, NCHD, C)
    row3 = row.reshape(NW, NBLK, BLK, C)
    col3 = col.reshape(NW, NBLK, BLK, C)
    x_pad = jnp.pad(x, ((0, NPAD - N), (0, 0)))
    dis, y = _prep(rowp, x_pad)
    z2 = _agg(row3, col3, y)
    return _mm(z2, dis.reshape(NPAD, 1), W, b.reshape(1, D))
